# baseline (speedup 1.0000x reference)
import jax
import jax.numpy as jnp
from jax import lax
from jax.experimental import pallas as pl
from jax.experimental.pallas import tpu as pltpu
from jax.experimental.pallas import tpu_sc as plsc

B = 16384
V = 1000000
GMF_D = 16
MLP_D = 32

NC = 2
NS = 16
NW = NC * NS
GRP = 16
S = 8


def _make_sc_gather(bb):
  bpw = bb // NW
  ngrp = bpw // GRP
  hstg = bpw // 2

  def body(u_hbm, i_hbm, ugt, igt, umt, imt,
           ug_out, ig_out, um_out, im_out,
           idx_u, idx_i, ug_o, ig_o, um_o, im_o,
           bug, big, bum, bim, sems):
    wid = lax.axis_index("s") * NC + lax.axis_index("c")
    base = wid * bpw
    pltpu.sync_copy(u_hbm.at[pl.ds(base, bpw)], idx_u)
    pltpu.sync_copy(i_hbm.at[pl.ds(base, bpw)], idx_i)
    rows16 = lax.iota(jnp.int32, 16)

    def fetch(slot, su, si):
      cu = pl.ds(pl.multiple_of(su, 128), 128)
      ci = pl.ds(pl.multiple_of(si, 128), 128)
      return [
          pltpu.async_copy(ugt.at[:, cu], bug[slot], sems[slot]),
          pltpu.async_copy(igt.at[:, ci], big[slot], sems[slot]),
          pltpu.async_copy(umt.at[:, cu], bum[slot], sems[slot]),
          pltpu.async_copy(imt.at[:, ci], bim[slot], sems[slot]),
      ]

    def drain(slot):
      pltpu.make_async_copy(ugt.at[:, pl.ds(0, 128)], bug[slot],
                            sems[slot]).wait()
      pltpu.make_async_copy(igt.at[:, pl.ds(0, 128)], big[slot],
                            sems[slot]).wait()
      pltpu.make_async_copy(umt.at[:, pl.ds(0, 128)], bum[slot],
                            sems[slot]).wait()
      pltpu.make_async_copy(imt.at[:, pl.ds(0, 128)], bim[slot],
                            sems[slot]).wait()

    vu0 = idx_u[pl.ds(0, GRP)]
    vi0 = idx_i[pl.ds(0, GRP)]
    for b in range(S):
      fetch(b, (vu0[b] >> 7) << 7, (vi0[b] >> 7) << 7)

    def group(g, carry):
      vu = idx_u[pl.ds(g * GRP, GRP)]
      vi = idx_i[pl.ds(g * GRP, GRP)]
      gn = jnp.minimum(g + 1, ngrp - 1)
      vun = idx_u[pl.ds(gn * GRP, GRP)]
      vin = idx_i[pl.ds(gn * GRP, GRP)]
      au = ((vu >> 7) << 7)
      ai = ((vi >> 7) << 7)
      aun = ((vun >> 7) << 7)
      ain = ((vin >> 7) << 7)
      ku = vu & 127
      ki = vi & 127
      for k in range(GRP):
        b = k % S
        drain(b)
        rr = jnp.broadcast_to((g * GRP + k) & (hstg - 1), (16,)).astype(
            jnp.int32)
        cku = jnp.broadcast_to(ku[k], (16,))
        cki = jnp.broadcast_to(ki[k], (16,))
        plsc.store_scatter(ug_o, [rows16, rr],
                           plsc.load_gather(bug[b], [rows16, cku]))
        plsc.store_scatter(ig_o, [rows16, rr],
                           plsc.load_gather(big[b], [rows16, cki]))
        plsc.store_scatter(um_o, [rows16, rr],
                           plsc.load_gather(bum[b], [rows16, cku]))
        plsc.store_scatter(um_o, [rows16 + 16, rr],
                           plsc.load_gather(bum[b], [rows16 + 16, cku]))
        plsc.store_scatter(im_o, [rows16, rr],
                           plsc.load_gather(bim[b], [rows16, cki]))
        plsc.store_scatter(im_o, [rows16 + 16, rr],
                           plsc.load_gather(bim[b], [rows16 + 16, cki]))
        if k + S < GRP:
          fetch(b, au[k + S], ai[k + S])
        else:
          fetch(b, aun[k + S - GRP], ain[k + S - GRP])

      @pl.when(g == ngrp // 2 - 1)
      def _flush_first_half():
        cols0 = pl.ds(base, hstg)
        pltpu.sync_copy(ug_o, ug_out.at[:, cols0])
        pltpu.sync_copy(ig_o, ig_out.at[:, cols0])
        pltpu.sync_copy(um_o, um_out.at[:, cols0])
        pltpu.sync_copy(im_o, im_out.at[:, cols0])

      return carry

    lax.fori_loop(0, ngrp, group, 0)
    for b in range(S):
      drain(b)
    cols = pl.ds(base + hstg, hstg)
    pltpu.sync_copy(ug_o, ug_out.at[:, cols])
    pltpu.sync_copy(ig_o, ig_out.at[:, cols])
    pltpu.sync_copy(um_o, um_out.at[:, cols])
    pltpu.sync_copy(im_o, im_out.at[:, cols])

  return pl.kernel(
      body,
      out_type=(
          jax.ShapeDtypeStruct((GMF_D, bb), jnp.float32),
          jax.ShapeDtypeStruct((GMF_D, bb), jnp.float32),
          jax.ShapeDtypeStruct((MLP_D, bb), jnp.float32),
          jax.ShapeDtypeStruct((MLP_D, bb), jnp.float32),
      ),
      mesh=plsc.VectorSubcoreMesh(core_axis_name="c", subcore_axis_name="s"),
      scratch_types=[
          pltpu.VMEM((bpw,), jnp.int32),
          pltpu.VMEM((bpw,), jnp.int32),
          pltpu.VMEM((GMF_D, hstg), jnp.float32),
          pltpu.VMEM((GMF_D, hstg), jnp.float32),
          pltpu.VMEM((MLP_D, hstg), jnp.float32),
          pltpu.VMEM((MLP_D, hstg), jnp.float32),
          [pltpu.VMEM((GMF_D, 128), jnp.float32)] * S,
          [pltpu.VMEM((GMF_D, 128), jnp.float32)] * S,
          [pltpu.VMEM((MLP_D, 128), jnp.float32)] * S,
          [pltpu.VMEM((MLP_D, 128), jnp.float32)] * S,
          [pltpu.SemaphoreType.DMA] * S,
      ],
      compiler_params=pltpu.CompilerParams(needs_layout_passes=False),
  )


BLK = 2048


def _tc_mlp_body(ug_ref, ig_ref, um_ref, im_ref,
                 w0_ref, b0_ref, w1_ref, b1_ref, wfc_ref, bfc_ref, o_ref):
  x = jnp.concatenate([um_ref[...], im_ref[...]], axis=0)
  cd = (((0,), (0,)), ((), ()))
  h = lax.dot_general(w0_ref[...], x, cd)
  h = jnp.maximum(h + b0_ref[...].T, 0.0)
  m = lax.dot_general(w1_ref[...], h, cd)
  m = jnp.maximum(m + b1_ref[...].T, 0.0)
  g = ug_ref[...] * ig_ref[...]
  z = jnp.concatenate([g, m], axis=0)
  logit = lax.dot_general(wfc_ref[...], z, cd) + bfc_ref[...]
  o_ref[...] = jax.nn.sigmoid(logit)


def _tc_mlp(ug, ig, um, im, w0, b0, w1, b1, wfc, bfc):
  bb = ug.shape[1]
  grid = (bb // BLK,)
  col = lambda c: (0, c)
  rep = lambda c: (0, 0)
  return pl.pallas_call(
      _tc_mlp_body,
      grid=grid,
      in_specs=[
          pl.BlockSpec((GMF_D, BLK), col),
          pl.BlockSpec((GMF_D, BLK), col),
          pl.BlockSpec((MLP_D, BLK), col),
          pl.BlockSpec((MLP_D, BLK), col),
          pl.BlockSpec((64, 32), rep),
          pl.BlockSpec((1, 32), rep),
          pl.BlockSpec((32, 16), rep),
          pl.BlockSpec((1, 16), rep),
          pl.BlockSpec((32, 1), rep),
          pl.BlockSpec((1, 1), rep),
      ],
      out_specs=pl.BlockSpec((1, BLK), col),
      out_shape=jax.ShapeDtypeStruct((1, bb), jnp.float32),
  )(ug, ig, um, im, w0, b0, w1, b1, wfc, bfc)


_sc_gather_half = _make_sc_gather(B // 2)


@jax.jit
def kernel(u, i, Ugmf, Igmf, Umlp, Imlp, W0, b0, W1, b1, Wfc, bfc):
  u32 = u.astype(jnp.int32)
  i32 = i.astype(jnp.int32)
  ugt, igt, umt, imt = Ugmf.T, Igmf.T, Umlp.T, Imlp.T
  w_args = (W0, b0.reshape(1, -1), W1, b1.reshape(1, -1),
            Wfc, bfc.reshape(1, 1))
  halves = []
  for h in range(2):
    sl = slice(h * (B // 2), (h + 1) * (B // 2))
    ug, ig, um, im = _sc_gather_half(u32[sl], i32[sl], ugt, igt, umt, imt)
    halves.append(_tc_mlp(ug, ig, um, im, *w_args))
  return jnp.concatenate(halves, axis=1).reshape(B, 1)


# speedup vs baseline: 1.0456x; 1.0456x over previous
import jax
import jax.numpy as jnp
from jax import lax
from jax.experimental import pallas as pl
from jax.experimental.pallas import tpu as pltpu
from jax.experimental.pallas import tpu_sc as plsc

B = 16384
V = 1000000
GMF_D = 16
MLP_D = 32

NC = 2
NS = 16
NW = NC * NS
BPW = B // NW
GRP = 16
NGRP = BPW // GRP
S = 8
HSTG = BPW // 2


def _sc_gather_body(u_hbm, i_hbm, ugt, igt, umt, imt,
                    ug_out, ig_out, um_out, im_out,
                    idx_u, idx_i, ug_o, ig_o, um_o, im_o,
                    bug, big, bum, bim, sems):
  wid = lax.axis_index("s") * NC + lax.axis_index("c")
  base = wid * BPW
  pltpu.sync_copy(u_hbm.at[pl.ds(base, BPW)], idx_u)
  pltpu.sync_copy(i_hbm.at[pl.ds(base, BPW)], idx_i)
  rows16 = lax.iota(jnp.int32, 16)

  def fetch(slot, su, si):
    cu = pl.ds(pl.multiple_of(su, 128), 128)
    ci = pl.ds(pl.multiple_of(si, 128), 128)
    return [
        pltpu.async_copy(ugt.at[:, cu], bug[slot], sems[slot]),
        pltpu.async_copy(igt.at[:, ci], big[slot], sems[slot]),
        pltpu.async_copy(umt.at[:, cu], bum[slot], sems[slot]),
        pltpu.async_copy(imt.at[:, ci], bim[slot], sems[slot]),
    ]

  def drain(slot):
    pltpu.make_async_copy(ugt.at[:, pl.ds(0, 128)], bug[slot],
                          sems[slot]).wait()
    pltpu.make_async_copy(igt.at[:, pl.ds(0, 128)], big[slot],
                          sems[slot]).wait()
    pltpu.make_async_copy(umt.at[:, pl.ds(0, 128)], bum[slot],
                          sems[slot]).wait()
    pltpu.make_async_copy(imt.at[:, pl.ds(0, 128)], bim[slot],
                          sems[slot]).wait()

  vu0 = idx_u[pl.ds(0, GRP)]
  vi0 = idx_i[pl.ds(0, GRP)]
  for b in range(S):
    fetch(b, (vu0[b] >> 7) << 7, (vi0[b] >> 7) << 7)

  def group(g, carry):
    vu = idx_u[pl.ds(g * GRP, GRP)]
    vi = idx_i[pl.ds(g * GRP, GRP)]
    gn = jnp.minimum(g + 1, NGRP - 1)
    vun = idx_u[pl.ds(gn * GRP, GRP)]
    vin = idx_i[pl.ds(gn * GRP, GRP)]
    au = ((vu >> 7) << 7)
    ai = ((vi >> 7) << 7)
    aun = ((vun >> 7) << 7)
    ain = ((vin >> 7) << 7)
    ku = vu & 127
    ki = vi & 127
    for k in range(GRP):
      b = k % S
      drain(b)
      rr = jnp.broadcast_to((g * GRP + k) & (HSTG - 1), (16,)).astype(
          jnp.int32)
      cku = jnp.broadcast_to(ku[k], (16,))
      cki = jnp.broadcast_to(ki[k], (16,))
      plsc.store_scatter(ug_o, [rows16, rr],
                         plsc.load_gather(bug[b], [rows16, cku]))
      plsc.store_scatter(ig_o, [rows16, rr],
                         plsc.load_gather(big[b], [rows16, cki]))
      plsc.store_scatter(um_o, [rows16, rr],
                         plsc.load_gather(bum[b], [rows16, cku]))
      plsc.store_scatter(um_o, [rows16 + 16, rr],
                         plsc.load_gather(bum[b], [rows16 + 16, cku]))
      plsc.store_scatter(im_o, [rows16, rr],
                         plsc.load_gather(bim[b], [rows16, cki]))
      plsc.store_scatter(im_o, [rows16 + 16, rr],
                         plsc.load_gather(bim[b], [rows16 + 16, cki]))
      if k + S < GRP:
        fetch(b, au[k + S], ai[k + S])
      else:
        fetch(b, aun[k + S - GRP], ain[k + S - GRP])

    @pl.when(g == NGRP // 2 - 1)
    def _flush_first_half():
      cols0 = pl.ds(base, HSTG)
      pltpu.sync_copy(ug_o, ug_out.at[:, cols0])
      pltpu.sync_copy(ig_o, ig_out.at[:, cols0])
      pltpu.sync_copy(um_o, um_out.at[:, cols0])
      pltpu.sync_copy(im_o, im_out.at[:, cols0])

    return carry

  lax.fori_loop(0, NGRP, group, 0)
  for b in range(S):
    drain(b)
  cols = pl.ds(base + HSTG, HSTG)
  pltpu.sync_copy(ug_o, ug_out.at[:, cols])
  pltpu.sync_copy(ig_o, ig_out.at[:, cols])
  pltpu.sync_copy(um_o, um_out.at[:, cols])
  pltpu.sync_copy(im_o, im_out.at[:, cols])


_sc_gather = pl.kernel(
    _sc_gather_body,
    out_type=(
        jax.ShapeDtypeStruct((GMF_D, B), jnp.float32),
        jax.ShapeDtypeStruct((GMF_D, B), jnp.float32),
        jax.ShapeDtypeStruct((MLP_D, B), jnp.float32),
        jax.ShapeDtypeStruct((MLP_D, B), jnp.float32),
    ),
    mesh=plsc.VectorSubcoreMesh(core_axis_name="c", subcore_axis_name="s"),
    scratch_types=[
        pltpu.VMEM((BPW,), jnp.int32),
        pltpu.VMEM((BPW,), jnp.int32),
        pltpu.VMEM((GMF_D, HSTG), jnp.float32),
        pltpu.VMEM((GMF_D, HSTG), jnp.float32),
        pltpu.VMEM((MLP_D, HSTG), jnp.float32),
        pltpu.VMEM((MLP_D, HSTG), jnp.float32),
        [pltpu.VMEM((GMF_D, 128), jnp.float32)] * S,
        [pltpu.VMEM((GMF_D, 128), jnp.float32)] * S,
        [pltpu.VMEM((MLP_D, 128), jnp.float32)] * S,
        [pltpu.VMEM((MLP_D, 128), jnp.float32)] * S,
        [pltpu.SemaphoreType.DMA] * S,
    ],
    compiler_params=pltpu.CompilerParams(needs_layout_passes=False),
)


BLK = 2048


def _tc_mlp_body(ug_ref, ig_ref, um_ref, im_ref,
                 w0_ref, b0_ref, w1_ref, b1_ref, wfc_ref, bfc_ref, o_ref):
  x = jnp.concatenate([um_ref[...], im_ref[...]], axis=0)
  cd = (((0,), (0,)), ((), ()))
  h = lax.dot_general(w0_ref[...], x, cd)
  h = jnp.maximum(h + b0_ref[...].T, 0.0)
  m = lax.dot_general(w1_ref[...], h, cd)
  m = jnp.maximum(m + b1_ref[...].T, 0.0)
  g = ug_ref[...] * ig_ref[...]
  z = jnp.concatenate([g, m], axis=0)
  logit = lax.dot_general(wfc_ref[...], z, cd) + bfc_ref[...]
  o_ref[...] = jax.nn.sigmoid(logit)


def _tc_mlp(ug, ig, um, im, w0, b0, w1, b1, wfc, bfc):
  grid = (B // BLK,)
  col = lambda c: (0, c)
  rep = lambda c: (0, 0)
  return pl.pallas_call(
      _tc_mlp_body,
      grid=grid,
      in_specs=[
          pl.BlockSpec((GMF_D, BLK), col),
          pl.BlockSpec((GMF_D, BLK), col),
          pl.BlockSpec((MLP_D, BLK), col),
          pl.BlockSpec((MLP_D, BLK), col),
          pl.BlockSpec((64, 32), rep),
          pl.BlockSpec((1, 32), rep),
          pl.BlockSpec((32, 16), rep),
          pl.BlockSpec((1, 16), rep),
          pl.BlockSpec((32, 1), rep),
          pl.BlockSpec((1, 1), rep),
      ],
      out_specs=pl.BlockSpec((1, BLK), col),
      out_shape=jax.ShapeDtypeStruct((1, B), jnp.float32),
  )(ug, ig, um, im, w0, b0, w1, b1, wfc, bfc)


@jax.jit
def kernel(u, i, Ugmf, Igmf, Umlp, Imlp, W0, b0, W1, b1, Wfc, bfc):
  ug, ig, um, im = _sc_gather(u.astype(jnp.int32), i.astype(jnp.int32),
                              Ugmf.T, Igmf.T, Umlp.T, Imlp.T)
  out = _tc_mlp(ug, ig, um, im,
                W0, b0.reshape(1, -1), W1, b1.reshape(1, -1),
                Wfc, bfc.reshape(1, 1))
  return out.reshape(B, 1)


# speedup vs baseline: 1.0551x; 1.0091x over previous
import jax
import jax.numpy as jnp
from jax import lax
from jax.experimental import pallas as pl
from jax.experimental.pallas import tpu as pltpu
from jax.experimental.pallas import tpu_sc as plsc

B = 16384
V = 1000000
GMF_D = 16
MLP_D = 32

NC = 2
NS = 16
NW = NC * NS
BPW = B // NW
GRP = 16
NGRP = BPW // GRP
S = 8
HSTG = BPW // 2


def _sc_gather_body(u_hbm, i_hbm, ugt, igt, umt, imt,
                    ug_out, ig_out, um_out, im_out,
                    idx_u, idx_i, ug_o, ig_o, um_o, im_o,
                    bug, big, bum, bim, sems):
  wid = lax.axis_index("s") * NC + lax.axis_index("c")
  base = wid * BPW
  pltpu.sync_copy(u_hbm.at[pl.ds(base, BPW)], idx_u)
  pltpu.sync_copy(i_hbm.at[pl.ds(base, BPW)], idx_i)
  rows16 = lax.iota(jnp.int32, 16)

  def fetch(slot, su, si):
    cu = pl.ds(pl.multiple_of(su, 128), 128)
    ci = pl.ds(pl.multiple_of(si, 128), 128)
    return [
        pltpu.async_copy(ugt.at[:, cu], bug[slot], sems[slot]),
        pltpu.async_copy(igt.at[:, ci], big[slot], sems[slot]),
        pltpu.async_copy(umt.at[:, cu], bum[slot], sems[slot]),
        pltpu.async_copy(imt.at[:, ci], bim[slot], sems[slot]),
    ]

  def drain(slot):
    pltpu.make_async_copy(ugt.at[:, pl.ds(0, 128)], bug[slot],
                          sems[slot]).wait()
    pltpu.make_async_copy(igt.at[:, pl.ds(0, 128)], big[slot],
                          sems[slot]).wait()
    pltpu.make_async_copy(umt.at[:, pl.ds(0, 128)], bum[slot],
                          sems[slot]).wait()
    pltpu.make_async_copy(imt.at[:, pl.ds(0, 128)], bim[slot],
                          sems[slot]).wait()

  vu0 = idx_u[pl.ds(0, GRP)]
  vi0 = idx_i[pl.ds(0, GRP)]
  for b in range(S):
    fetch(b, (vu0[b] >> 7) << 7, (vi0[b] >> 7) << 7)

  def group(g, carry):
    vu = idx_u[pl.ds(g * GRP, GRP)]
    vi = idx_i[pl.ds(g * GRP, GRP)]
    gn = jnp.minimum(g + 1, NGRP - 1)
    vun = idx_u[pl.ds(gn * GRP, GRP)]
    vin = idx_i[pl.ds(gn * GRP, GRP)]
    au = ((vu >> 7) << 7)
    ai = ((vi >> 7) << 7)
    aun = ((vun >> 7) << 7)
    ain = ((vin >> 7) << 7)
    ku = vu & 127
    ki = vi & 127
    for k in range(GRP):
      b = k % S
      drain(b)
      rr = jnp.broadcast_to((g * GRP + k) & (HSTG - 1), (16,)).astype(
          jnp.int32)
      cku = jnp.broadcast_to(ku[k], (16,))
      cki = jnp.broadcast_to(ki[k], (16,))
      plsc.store_scatter(ug_o, [rows16, rr],
                         plsc.load_gather(bug[b], [rows16, cku]))
      plsc.store_scatter(ig_o, [rows16, rr],
                         plsc.load_gather(big[b], [rows16, cki]))
      plsc.store_scatter(um_o, [rows16, rr],
                         plsc.load_gather(bum[b], [rows16, cku]))
      plsc.store_scatter(um_o, [rows16 + 16, rr],
                         plsc.load_gather(bum[b], [rows16 + 16, cku]))
      plsc.store_scatter(im_o, [rows16, rr],
                         plsc.load_gather(bim[b], [rows16, cki]))
      plsc.store_scatter(im_o, [rows16 + 16, rr],
                         plsc.load_gather(bim[b], [rows16 + 16, cki]))
      if k + S < GRP:
        fetch(b, au[k + S], ai[k + S])
      else:
        fetch(b, aun[k + S - GRP], ain[k + S - GRP])

    @pl.when(g == NGRP // 2 - 1)
    def _flush_first_half():
      cols0 = pl.ds(base, HSTG)
      pltpu.sync_copy(ug_o, ug_out.at[:, cols0])
      pltpu.sync_copy(ig_o, ig_out.at[:, cols0])
      pltpu.sync_copy(um_o, um_out.at[:, cols0])
      pltpu.sync_copy(im_o, im_out.at[:, cols0])

    return carry

  lax.fori_loop(0, NGRP, group, 0)
  for b in range(S):
    drain(b)
  cols = pl.ds(base + HSTG, HSTG)
  pltpu.sync_copy(ug_o, ug_out.at[:, cols])
  pltpu.sync_copy(ig_o, ig_out.at[:, cols])
  pltpu.sync_copy(um_o, um_out.at[:, cols])
  pltpu.sync_copy(im_o, im_out.at[:, cols])


_sc_gather = pl.kernel(
    _sc_gather_body,
    out_type=(
        jax.ShapeDtypeStruct((GMF_D, B), jnp.float32),
        jax.ShapeDtypeStruct((GMF_D, B), jnp.float32),
        jax.ShapeDtypeStruct((MLP_D, B), jnp.float32),
        jax.ShapeDtypeStruct((MLP_D, B), jnp.float32),
    ),
    mesh=plsc.VectorSubcoreMesh(core_axis_name="c", subcore_axis_name="s"),
    scratch_types=[
        pltpu.VMEM((BPW,), jnp.int32),
        pltpu.VMEM((BPW,), jnp.int32),
        pltpu.VMEM((GMF_D, HSTG), jnp.float32),
        pltpu.VMEM((GMF_D, HSTG), jnp.float32),
        pltpu.VMEM((MLP_D, HSTG), jnp.float32),
        pltpu.VMEM((MLP_D, HSTG), jnp.float32),
        [pltpu.VMEM((GMF_D, 128), jnp.float32)] * S,
        [pltpu.VMEM((GMF_D, 128), jnp.float32)] * S,
        [pltpu.VMEM((MLP_D, 128), jnp.float32)] * S,
        [pltpu.VMEM((MLP_D, 128), jnp.float32)] * S,
        [pltpu.SemaphoreType.DMA] * S,
    ],
    compiler_params=pltpu.CompilerParams(needs_layout_passes=False),
)


BLK = 4096


def _tc_mlp_body(ug_ref, ig_ref, um_ref, im_ref,
                 w0_ref, b0_ref, w1_ref, b1_ref, wfc_ref, bfc_ref, o_ref):
  x = jnp.concatenate([um_ref[...], im_ref[...]], axis=0)
  cd = (((0,), (0,)), ((), ()))
  h = lax.dot_general(w0_ref[...], x, cd)
  h = jnp.maximum(h + b0_ref[...].T, 0.0)
  m = lax.dot_general(w1_ref[...], h, cd)
  m = jnp.maximum(m + b1_ref[...].T, 0.0)
  g = ug_ref[...] * ig_ref[...]
  z = jnp.concatenate([g, m], axis=0)
  logit = lax.dot_general(wfc_ref[...], z, cd) + bfc_ref[...]
  o_ref[...] = jax.nn.sigmoid(logit)


def _tc_mlp(ug, ig, um, im, w0, b0, w1, b1, wfc, bfc):
  grid = (B // BLK,)
  col = lambda c: (0, c)
  rep = lambda c: (0, 0)
  return pl.pallas_call(
      _tc_mlp_body,
      grid=grid,
      in_specs=[
          pl.BlockSpec((GMF_D, BLK), col),
          pl.BlockSpec((GMF_D, BLK), col),
          pl.BlockSpec((MLP_D, BLK), col),
          pl.BlockSpec((MLP_D, BLK), col),
          pl.BlockSpec((64, 32), rep),
          pl.BlockSpec((1, 32), rep),
          pl.BlockSpec((32, 16), rep),
          pl.BlockSpec((1, 16), rep),
          pl.BlockSpec((32, 1), rep),
          pl.BlockSpec((1, 1), rep),
      ],
      out_specs=pl.BlockSpec((1, BLK), col),
      out_shape=jax.ShapeDtypeStruct((1, B), jnp.float32),
  )(ug, ig, um, im, w0, b0, w1, b1, wfc, bfc)


@jax.jit
def kernel(u, i, Ugmf, Igmf, Umlp, Imlp, W0, b0, W1, b1, Wfc, bfc):
  ug, ig, um, im = _sc_gather(u.astype(jnp.int32), i.astype(jnp.int32),
                              Ugmf.T, Igmf.T, Umlp.T, Imlp.T)
  out = _tc_mlp(ug, ig, um, im,
                W0, b0.reshape(1, -1), W1, b1.reshape(1, -1),
                Wfc, bfc.reshape(1, 1))
  return out.reshape(B, 1)
